# baseline (speedup 1.0000x reference)
import functools

import jax
import jax.numpy as jnp
from jax import lax
from jax.experimental import pallas as pl
from jax.experimental.pallas import tpu as pltpu
from jax.experimental.pallas import tpu_sc as plsc

N_NODES = 50000
NTYPES_NODE = 95
DIM_NODE = 128
N_EDGES = 800000
DEGREE_RBF = 16
DIM_EDGE = 64

_NW = 32
_B_W = 1568
_CH = 392
_N_CH = _B_W // _CH


@functools.partial(
    pl.kernel,
    out_type=jax.ShapeDtypeStruct((N_NODES, DIM_NODE), jnp.float32),
    mesh=plsc.VectorSubcoreMesh(core_axis_name="c", subcore_axis_name="s"),
    scratch_types=[
        pltpu.VMEM((_B_W,), jnp.int32),
        pltpu.VMEM((_CH, DIM_NODE), jnp.float32),
        pltpu.VMEM((_CH, DIM_NODE), jnp.float32),
        pltpu.VMEM_SHARED((NTYPES_NODE, DIM_NODE), jnp.float32),
        pltpu.SemaphoreType.DMA,
        pltpu.SemaphoreType.DMA,
        pltpu.SemaphoreType.DMA,
    ],
)
def _sc_gather(idx_hbm, table_hbm, out_hbm, idx_v, rows0, rows1, table_s,
               sem_g, sem0, sem1):
    sid = lax.axis_index("s")
    wid = sid * 2 + lax.axis_index("c")
    base = jnp.minimum(wid * _B_W, N_NODES - _B_W)
    @pl.when(sid == 0)
    def _():
        pltpu.sync_copy(table_hbm, table_s)
    pltpu.sync_copy(idx_hbm.at[pl.ds(base, _B_W)], idx_v)
    plsc.subcore_barrier()
    bufs = (rows0, rows1)
    sems = (sem0, sem1)
    cps = []
    for ch in range(_N_CH):
        b = ch % 2
        if ch >= 2:
            cps[ch - 2].wait()
        pltpu.async_copy(
            table_s.at[idx_v.at[pl.ds(ch * _CH, _CH)]], bufs[b], sem_g
        ).wait()
        cps.append(
            pltpu.async_copy(
                bufs[b], out_hbm.at[pl.ds(base + ch * _CH, _CH)], sems[b]
            )
        )
    cps[-2].wait()
    cps[-1].wait()


_BN = 16000


def _mlp_body(x_ref, w_ref, b_ref, o_ref):
    o_ref[...] = jnp.maximum(
        jnp.dot(w_ref[...], x_ref[...], preferred_element_type=jnp.float32)
        + b_ref[...],
        0.0,
    )


def _edge_mlp(edge_attr, W_e, b_e):
    out_t = pl.pallas_call(
        _mlp_body,
        grid=(N_EDGES // _BN,),
        in_specs=[
            pl.BlockSpec((DEGREE_RBF, _BN), lambda i: (0, i)),
            pl.BlockSpec((DIM_EDGE, DEGREE_RBF), lambda i: (0, 0)),
            pl.BlockSpec((DIM_EDGE, 1), lambda i: (0, 0)),
        ],
        out_specs=pl.BlockSpec((DIM_EDGE, _BN), lambda i: (0, i)),
        out_shape=jax.ShapeDtypeStruct((DIM_EDGE, N_EDGES), jnp.float32),
    )(edge_attr.T, W_e.T, b_e.reshape(DIM_EDGE, 1))
    return out_t.T


def kernel(node_attr, edge_attr, state_attr, node_table, W_e, b_e):
    node_feat = _sc_gather(node_attr.astype(jnp.int32), node_table)
    edge_feat = _edge_mlp(edge_attr, W_e, b_e)
    return (node_feat, edge_feat)


# speedup vs baseline: 1.0963x; 1.0963x over previous
import functools

import jax
import jax.numpy as jnp
from jax import lax
from jax.experimental import pallas as pl
from jax.experimental.pallas import tpu as pltpu
from jax.experimental.pallas import tpu_sc as plsc

N_NODES = 50000
NTYPES_NODE = 95
DIM_NODE = 128
N_EDGES = 800000
DEGREE_RBF = 16
DIM_EDGE = 64

_NW = 32
_B_W = 1568
_CH = 392
_N_CH = _B_W // _CH


@functools.partial(
    pl.kernel,
    out_type=jax.ShapeDtypeStruct((N_NODES, DIM_NODE), jnp.float32),
    mesh=plsc.VectorSubcoreMesh(core_axis_name="c", subcore_axis_name="s"),
    scratch_types=[
        pltpu.VMEM((_B_W,), jnp.int32),
        pltpu.VMEM((_CH, DIM_NODE), jnp.float32),
        pltpu.VMEM((_CH, DIM_NODE), jnp.float32),
        pltpu.VMEM_SHARED((NTYPES_NODE, DIM_NODE), jnp.float32),
        pltpu.SemaphoreType.DMA,
        pltpu.SemaphoreType.DMA,
        pltpu.SemaphoreType.DMA,
    ],
)
def _sc_gather(idx_hbm, table_hbm, out_hbm, idx_v, rows0, rows1, table_s,
               sem_g, sem0, sem1):
    sid = lax.axis_index("s")
    wid = sid * 2 + lax.axis_index("c")
    base = jnp.minimum(wid * _B_W, N_NODES - _B_W)
    @pl.when(sid == 0)
    def _():
        pltpu.sync_copy(table_hbm, table_s)
    pltpu.sync_copy(idx_hbm.at[pl.ds(base, _B_W)], idx_v)
    plsc.subcore_barrier()
    bufs = (rows0, rows1)
    sems = (sem0, sem1)
    cps = []
    for ch in range(_N_CH):
        b = ch % 2
        if ch >= 2:
            cps[ch - 2].wait()
        pltpu.async_copy(
            table_s.at[idx_v.at[pl.ds(ch * _CH, _CH)]], bufs[b], sem_g
        ).wait()
        cps.append(
            pltpu.async_copy(
                bufs[b], out_hbm.at[pl.ds(base + ch * _CH, _CH)], sems[b]
            )
        )
    cps[-2].wait()
    cps[-1].wait()


_BN = 80000


def _mlp_body(x_ref, w_ref, b_ref, o_ref):
    o_ref[...] = jnp.maximum(
        jnp.dot(w_ref[...], x_ref[...], preferred_element_type=jnp.float32)
        + b_ref[...],
        0.0,
    )


def _edge_mlp(edge_attr, W_e, b_e):
    out_t = pl.pallas_call(
        _mlp_body,
        grid=(N_EDGES // _BN,),
        in_specs=[
            pl.BlockSpec((DEGREE_RBF, _BN), lambda i: (0, i)),
            pl.BlockSpec((DIM_EDGE, DEGREE_RBF), lambda i: (0, 0)),
            pl.BlockSpec((DIM_EDGE, 1), lambda i: (0, 0)),
        ],
        out_specs=pl.BlockSpec((DIM_EDGE, _BN), lambda i: (0, i)),
        out_shape=jax.ShapeDtypeStruct((DIM_EDGE, N_EDGES), jnp.float32),
    )(edge_attr.T, W_e.T, b_e.reshape(DIM_EDGE, 1))
    return out_t.T


def kernel(node_attr, edge_attr, state_attr, node_table, W_e, b_e):
    node_feat = _sc_gather(node_attr.astype(jnp.int32), node_table)
    edge_feat = _edge_mlp(edge_attr, W_e, b_e)
    return (node_feat, edge_feat)


# speedup vs baseline: 1.1043x; 1.0073x over previous
import functools

import jax
import jax.numpy as jnp
from jax import lax
from jax.experimental import pallas as pl
from jax.experimental.pallas import tpu as pltpu
from jax.experimental.pallas import tpu_sc as plsc

N_NODES = 50000
NTYPES_NODE = 95
DIM_NODE = 128
N_EDGES = 800000
DEGREE_RBF = 16
DIM_EDGE = 64

_NW = 32
_B_W = 1568
_CH = 392
_N_CH = _B_W // _CH


@functools.partial(
    pl.kernel,
    out_type=jax.ShapeDtypeStruct((N_NODES, DIM_NODE), jnp.float32),
    mesh=plsc.VectorSubcoreMesh(core_axis_name="c", subcore_axis_name="s"),
    scratch_types=[
        pltpu.VMEM((_B_W,), jnp.int32),
        pltpu.VMEM((_CH, DIM_NODE), jnp.float32),
        pltpu.VMEM((_CH, DIM_NODE), jnp.float32),
        pltpu.VMEM_SHARED((NTYPES_NODE, DIM_NODE), jnp.float32),
        pltpu.SemaphoreType.DMA,
        pltpu.SemaphoreType.DMA,
        pltpu.SemaphoreType.DMA,
    ],
)
def _sc_gather(idx_hbm, table_hbm, out_hbm, idx_v, rows0, rows1, table_s,
               sem_g, sem0, sem1):
    sid = lax.axis_index("s")
    wid = sid * 2 + lax.axis_index("c")
    base = jnp.minimum(wid * _B_W, N_NODES - _B_W)
    @pl.when(sid == 0)
    def _():
        pltpu.sync_copy(table_hbm, table_s)
    pltpu.sync_copy(idx_hbm.at[pl.ds(base, _B_W)], idx_v)
    plsc.subcore_barrier()
    bufs = (rows0, rows1)
    sems = (sem0, sem1)
    cps = []
    for ch in range(_N_CH):
        b = ch % 2
        if ch >= 2:
            cps[ch - 2].wait()
        pltpu.async_copy(
            table_s.at[idx_v.at[pl.ds(ch * _CH, _CH)]], bufs[b], sem_g
        ).wait()
        cps.append(
            pltpu.async_copy(
                bufs[b], out_hbm.at[pl.ds(base + ch * _CH, _CH)], sems[b]
            )
        )
    cps[-2].wait()
    cps[-1].wait()


_BN = 80000


def _mlp_body(x_ref, w_ref, b_ref, o_ref):
    wtx = lax.dot_general(
        w_ref[...], x_ref[...],
        (((0,), (0,)), ((), ())),
        preferred_element_type=jnp.float32,
    )
    o_ref[...] = jnp.maximum(wtx + b_ref[...], 0.0)


def _edge_mlp(edge_attr, W_e, b_e):
    out_t = pl.pallas_call(
        _mlp_body,
        grid=(N_EDGES // _BN,),
        in_specs=[
            pl.BlockSpec((DEGREE_RBF, _BN), lambda i: (0, i)),
            pl.BlockSpec((DEGREE_RBF, DIM_EDGE), lambda i: (0, 0)),
            pl.BlockSpec((DIM_EDGE, 1), lambda i: (0, 0)),
        ],
        out_specs=pl.BlockSpec((DIM_EDGE, _BN), lambda i: (0, i)),
        out_shape=jax.ShapeDtypeStruct((DIM_EDGE, N_EDGES), jnp.float32),
    )(edge_attr.T, W_e, b_e.reshape(DIM_EDGE, 1))
    return out_t.T


def kernel(node_attr, edge_attr, state_attr, node_table, W_e, b_e):
    node_feat = _sc_gather(node_attr.astype(jnp.int32), node_table)
    edge_feat = _edge_mlp(edge_attr, W_e, b_e)
    return (node_feat, edge_feat)


# speedup vs baseline: 1.1167x; 1.0112x over previous
import functools

import jax
import jax.numpy as jnp
from jax import lax
from jax.experimental import pallas as pl
from jax.experimental.pallas import tpu as pltpu
from jax.experimental.pallas import tpu_sc as plsc

N_NODES = 50000
NTYPES_NODE = 95
DIM_NODE = 128
N_EDGES = 800000
DEGREE_RBF = 16
DIM_EDGE = 64

_NW = 32
_B_W = 1568
_CH = 392
_N_CH = _B_W // _CH


@functools.partial(
    pl.kernel,
    out_type=jax.ShapeDtypeStruct((N_NODES, DIM_NODE), jnp.float32),
    mesh=plsc.VectorSubcoreMesh(core_axis_name="c", subcore_axis_name="s"),
    scratch_types=[
        pltpu.VMEM((_B_W,), jnp.int32),
        pltpu.VMEM((_CH, DIM_NODE), jnp.float32),
        pltpu.VMEM((_CH, DIM_NODE), jnp.float32),
        pltpu.VMEM_SHARED((NTYPES_NODE, DIM_NODE), jnp.float32),
        pltpu.SemaphoreType.DMA,
        pltpu.SemaphoreType.DMA,
        pltpu.SemaphoreType.DMA,
    ],
)
def _sc_gather(idx_hbm, table_hbm, out_hbm, idx_v, rows0, rows1, table_s,
               sem_g, sem0, sem1):
    sid = lax.axis_index("s")
    wid = sid * 2 + lax.axis_index("c")
    base = jnp.minimum(wid * _B_W, N_NODES - _B_W)
    @pl.when(sid == 0)
    def _():
        pltpu.sync_copy(table_hbm, table_s)
    pltpu.sync_copy(idx_hbm.at[pl.ds(base, _B_W)], idx_v)
    plsc.subcore_barrier()
    bufs = (rows0, rows1)
    sems = (sem0, sem1)
    cps = []
    for ch in range(_N_CH):
        b = ch % 2
        if ch >= 2:
            cps[ch - 2].wait()
        pltpu.async_copy(
            table_s.at[idx_v.at[pl.ds(ch * _CH, _CH)]], bufs[b], sem_g
        ).wait()
        cps.append(
            pltpu.async_copy(
                bufs[b], out_hbm.at[pl.ds(base + ch * _CH, _CH)], sems[b]
            )
        )
    cps[-2].wait()
    cps[-1].wait()


_BN = 80000


def _mlp_body(x_ref, w_ref, b_ref, o_ref):
    wtx = lax.dot_general(
        w_ref[...], x_ref[...],
        (((0,), (0,)), ((), ())),
        preferred_element_type=jnp.float32,
    )
    o_ref[...] = jnp.maximum(wtx + b_ref[...].T, 0.0)


def _edge_mlp(edge_attr, W_e, b_e):
    out_t = pl.pallas_call(
        _mlp_body,
        grid=(N_EDGES // _BN,),
        in_specs=[
            pl.BlockSpec((DEGREE_RBF, _BN), lambda i: (0, i)),
            pl.BlockSpec((DEGREE_RBF, DIM_EDGE), lambda i: (0, 0)),
            pl.BlockSpec((1, DIM_EDGE), lambda i: (0, 0)),
        ],
        out_specs=pl.BlockSpec((DIM_EDGE, _BN), lambda i: (0, i)),
        out_shape=jax.ShapeDtypeStruct((DIM_EDGE, N_EDGES), jnp.float32),
    )(edge_attr.T, W_e, b_e.reshape(1, DIM_EDGE))
    return out_t.T


def kernel(node_attr, edge_attr, state_attr, node_table, W_e, b_e):
    node_feat = _sc_gather(node_attr.astype(jnp.int32), node_table)
    edge_feat = _edge_mlp(edge_attr, W_e, b_e)
    return (node_feat, edge_feat)
